# Initial kernel scaffold; baseline (speedup 1.0000x reference)
import jax
import jax.numpy as jnp
from jax.experimental import pallas as pl


def kernel(x, edge_index, conv_w, conv_b, lin_res_w, lin_res_b, bn_w, bn_b):
    raise NotImplementedError("write your pallas kernel here")


# speedup vs baseline: 19.1817x; 19.1817x over previous
import functools

import jax
import jax.numpy as jnp
from jax import lax
from jax.experimental import pallas as pl
from jax.experimental.pallas import tpu as pltpu
from jax.experimental.pallas import tpu_sc as plsc

N = 10000
E = 320000
D = 128
NPAD = 10240
CH = 128
NCHUNK = E // CH
NW = 32
CPW = (NCHUNK + NW - 1) // NW
ROWS_PER_TILE = NPAD // 16
TAU = 2.0
V_TH = 1.0
EPS = 1e-5

_mesh = plsc.VectorSubcoreMesh(
    core_axis_name="c", subcore_axis_name="s", num_cores=2, num_subcores=16)


def _zero_vmem_2d(ref, nrows):
    z = jnp.zeros((16,), jnp.float32)

    def body(i, _):
        for m in range(8):
            ref[i, pl.ds(m * 16, 16)] = z
        return 0

    lax.fori_loop(0, nrows, body, 0)


def _zero_vmem_1d(ref, n):
    z = jnp.zeros((16,), jnp.float32)

    def body(i, _):
        ref[pl.ds(i * 16, 16)] = z
        return 0

    lax.fori_loop(0, n // 16, body, 0)


@functools.partial(
    pl.kernel,
    out_type=jax.ShapeDtypeStruct((2, NPAD), jnp.float32),
    mesh=_mesh,
    scratch_types=[
        pltpu.VMEM((CH,), jnp.int32),
        pltpu.VMEM((CH,), jnp.float32),
        pltpu.VMEM((ROWS_PER_TILE,), jnp.float32),
        pltpu.VMEM_SHARED((NPAD,), jnp.float32),
    ],
)
def _deg_kernel(col_hbm, out_hbm, cbuf, ones_v, zbuf, hist_sh):
    c = lax.axis_index("c")
    s = lax.axis_index("s")
    wid = s * 2 + c

    _zero_vmem_1d(zbuf, ROWS_PER_TILE)
    one = jnp.ones((16,), jnp.float32)
    for m in range(CH // 16):
        ones_v[pl.ds(m * 16, 16)] = one
    pltpu.sync_copy(zbuf, hist_sh.at[pl.ds(s * ROWS_PER_TILE, ROWS_PER_TILE)])
    plsc.subcore_barrier()

    def body(i, _):
        j = wid + i * NW

        @pl.when(j < NCHUNK)
        def _():
            pltpu.sync_copy(col_hbm.at[pl.ds(j * CH, CH)], cbuf)
            pltpu.sync_copy(ones_v, hist_sh.at[cbuf], add=True)

        return 0

    lax.fori_loop(0, CPW, body, 0)
    plsc.subcore_barrier()
    pltpu.sync_copy(hist_sh.at[pl.ds(s * ROWS_PER_TILE, ROWS_PER_TILE)],
                    out_hbm.at[c, pl.ds(s * ROWS_PER_TILE, ROWS_PER_TILE)])


def _proj_body(x_ref, w_ref, degc_ref, g_ref, sfac_ref, msk_ref):
    deg = degc_ref[:, 0:1] + degc_ref[:, 1:2]
    dinv = jnp.where(deg > 0, 1.0 / jnp.sqrt(jnp.maximum(deg, 1e-12)), 0.0)
    sfac_ref[...] = dinv / jnp.maximum(deg, 1.0)
    msk_ref[...] = (deg > 0).astype(jnp.float32)
    h = lax.dot_general(x_ref[...], w_ref[...], (((1,), (1,)), ((), ())),
                        preferred_element_type=jnp.float32)
    g_ref[...] = h * dinv[:N, :]


_proj = pl.pallas_call(
    _proj_body,
    out_shape=(
        jax.ShapeDtypeStruct((N, D), jnp.float32),
        jax.ShapeDtypeStruct((NPAD, 1), jnp.float32),
        jax.ShapeDtypeStruct((NPAD, 1), jnp.float32),
    ),
)


@functools.partial(
    pl.kernel,
    out_type=jax.ShapeDtypeStruct((2, NPAD, D), jnp.float32),
    mesh=_mesh,
    scratch_types=[
        pltpu.VMEM((CH,), jnp.int32),
        pltpu.VMEM((CH,), jnp.int32),
        pltpu.VMEM((CH, D), jnp.float32),
        pltpu.VMEM((64, D), jnp.float32),
        pltpu.VMEM_SHARED((NPAD, D), jnp.float32),
        pltpu.SemaphoreType.DMA,
    ],
)
def _scatter_kernel(g_hbm, row_hbm, col_hbm, out_hbm,
                    rbuf, cbuf, rows, zbuf, agg_sh, sem):
    c = lax.axis_index("c")
    s = lax.axis_index("s")
    wid = s * 2 + c

    _zero_vmem_2d(zbuf, 64)
    for k in range(ROWS_PER_TILE // 64):
        pltpu.sync_copy(zbuf, agg_sh.at[pl.ds(s * ROWS_PER_TILE + k * 64, 64)])
    plsc.subcore_barrier()

    def body(i, _):
        j = wid + i * NW

        @pl.when(j < NCHUNK)
        def _():
            base = j * CH
            pltpu.sync_copy(row_hbm.at[pl.ds(base, CH)], rbuf)
            pltpu.sync_copy(col_hbm.at[pl.ds(base, CH)], cbuf)
            pltpu.async_copy(g_hbm.at[rbuf], rows, sem).wait()
            pltpu.sync_copy(rows, agg_sh.at[cbuf], add=True)

        return 0

    lax.fori_loop(0, CPW, body, 0)
    plsc.subcore_barrier()

    for k in range(ROWS_PER_TILE // 64):
        r0 = s * ROWS_PER_TILE + k * 64
        pltpu.sync_copy(agg_sh.at[pl.ds(r0, 64)],
                        out_hbm.at[c, pl.ds(r0, 64), :])


def _epi_body(aggp_ref, sfac_ref, msk_ref, cb_ref, bnw_ref, bnb_ref, out_ref):
    a = aggp_ref[0, :N, :] + aggp_ref[1, :N, :]
    out = a * sfac_ref[:N, :] + msk_ref[:N, :] * cb_ref[...]
    mean = jnp.mean(out, axis=0, keepdims=True)
    var = jnp.mean((out - mean) * (out - mean), axis=0, keepdims=True)
    y = (out - mean) / jnp.sqrt(var + EPS) * bnw_ref[...] + bnb_ref[...]
    out_ref[...] = (y / TAU >= V_TH).astype(jnp.float32)


_epilogue = pl.pallas_call(
    _epi_body,
    out_shape=jax.ShapeDtypeStruct((N, D), jnp.float32),
)


def kernel(x, edge_index, conv_w, conv_b, lin_res_w, lin_res_b, bn_w, bn_b):
    del lin_res_w, lin_res_b
    row = edge_index[0].astype(jnp.int32)
    col = edge_index[1].astype(jnp.int32)
    degp = _deg_kernel(col)
    degc = jnp.transpose(degp)
    g, sfac, msk = _proj(x, conv_w, degc)
    aggp = _scatter_kernel(g, row, col)
    spike = _epilogue(aggp, sfac, msk,
                      conv_b.reshape(1, D),
                      bn_w.reshape(1, D), bn_b.reshape(1, D))
    return spike
